# Initial kernel scaffold; baseline (speedup 1.0000x reference)
import jax
import jax.numpy as jnp
from jax.experimental import pallas as pl


def kernel(params, adj_src, adj_dst, trna_ids, disease_ids):
    raise NotImplementedError("write your pallas kernel here")


# speedup vs baseline: 7.8827x; 7.8827x over previous
import functools

import jax
import jax.numpy as jnp
from jax import lax
from jax.experimental import pallas as pl
from jax.experimental.pallas import tpu as pltpu
from jax.experimental.pallas import tpu_sc as plsc

N = 10000
D = 128
E = 320000
B = 4096
LEAK = 0.2
TEMP = 0.5

NP = 10240
RB = 1024
NBLK = NP // RB

NC = 2
NS = 16
NW = NC * NS
CHUNK = 128
IDXR = CHUNK // 128
NCHUNK = E // CHUNK
SLAB = NP // NS

def _sc_mesh():
    return plsc.VectorSubcoreMesh(core_axis_name="c", subcore_axis_name="s",
                                  num_cores=NC, num_subcores=NS)


def _edge_body(src2_hbm, tgt2_hbm, s1_hbm, s2_hbm, hj_hbm, zr_hbm, zd_hbm,
               accr_hbm, accd_hbm,
               s1_v, s2_v, srci_v, tgti_v, rows_v, w_v, sem,
               sh_rows, sh_den):
    cid = lax.axis_index("c")
    sid = lax.axis_index("s")
    wid = sid * NC + cid

    pltpu.sync_copy(s1_hbm, s1_v)
    pltpu.sync_copy(s2_hbm, s2_v)
    base = sid * SLAB
    pltpu.sync_copy(zr_hbm, sh_rows.at[pl.ds(base, SLAB)])
    pltpu.sync_copy(zd_hbm, sh_den.at[pl.ds(base, SLAB)])
    plsc.subcore_barrier()

    nch = (NCHUNK - wid + NW - 1) // NW

    def chunk_body(j, carry):
        chunk = wid + NW * j
        rbase = chunk * IDXR
        pltpu.sync_copy(src2_hbm.at[pl.ds(rbase, IDXR)], srci_v)
        pltpu.sync_copy(tgt2_hbm.at[pl.ds(rbase, IDXR)], tgti_v)
        cps = [pltpu.async_copy(hj_hbm.at[tgti_v.at[jj]],
                                rows_v.at[pl.ds(jj * 128, 128)], sem)
               for jj in range(IDXR)]
        for jj in range(IDXR):
            def score_body(l, c, jj=jj):
                srcv = srci_v[jj, pl.ds(l * 16, 16)]
                tgtv = tgti_v[jj, pl.ds(l * 16, 16)]
                e = (plsc.load_gather(s1_v, [srcv]) +
                     plsc.load_gather(s2_v, [tgtv]))
                e = jnp.where(e >= 0.0, e, e * LEAK)
                e = jnp.clip(e, -30.0, 30.0)
                w_v[pl.ds(jj * 128 + l * 16, 16)] = jnp.exp(e)
                return c
            lax.fori_loop(0, 8, score_body, 0, unroll=True)
        for cp in cps:
            cp.wait()

        def scale_body(r, c):
            wspl = plsc.load_gather(w_v, [jnp.full((16,), 0, jnp.int32) + r])
            for cc in range(8):
                rows_v[r, pl.ds(cc * 16, 16)] = (
                    rows_v[r, pl.ds(cc * 16, 16)] * wspl)
            return c
        lax.fori_loop(0, CHUNK, scale_body, 0)

        for jj in range(IDXR):
            pltpu.sync_copy(rows_v.at[pl.ds(jj * 128, 128)],
                            sh_rows.at[srci_v.at[jj]], add=True)
            pltpu.sync_copy(w_v.at[pl.ds(jj * 128, 128)],
                            sh_den.at[srci_v.at[jj]], add=True)
        return carry

    lax.fori_loop(0, nch, chunk_body, 0)

    plsc.subcore_barrier()
    pltpu.sync_copy(sh_rows.at[pl.ds(base, SLAB)],
                    accr_hbm.at[cid, pl.ds(base, SLAB)])
    pltpu.sync_copy(sh_den.at[pl.ds(base, SLAB)],
                    accd_hbm.at[cid, pl.ds(base, SLAB)])


@functools.cache
def _edge_call_build():
  return pl.kernel(
    _edge_body,
    out_type=(jax.ShapeDtypeStruct((NC, NP, D), jnp.float32),
              jax.ShapeDtypeStruct((NC, NP), jnp.float32)),
    mesh=_sc_mesh(),
    scratch_types=(
        pltpu.VMEM((NP,), jnp.float32),
        pltpu.VMEM((NP,), jnp.float32),
        pltpu.VMEM((IDXR, 128), jnp.int32),
        pltpu.VMEM((IDXR, 128), jnp.int32),
        pltpu.VMEM((CHUNK, D), jnp.float32),
        pltpu.VMEM((CHUNK,), jnp.float32),
        pltpu.SemaphoreType.DMA,
        pltpu.VMEM_SHARED((NP, D), jnp.float32),
        pltpu.VMEM_SHARED((NP,), jnp.float32),
    ),
    compiler_params=pltpu.CompilerParams(needs_layout_passes=False),
  )


def _edge_call(*args):
    return _edge_call_build()(*args)


BPW = B // NW


def _select_body(sgt_hbm, hgt_hbm, sgd_hbm, hgd_hbm, tid_hbm, did_hbm,
                 selt_hbm, seld_hbm,
                 idx_v, a_v, b_v, sem):
    cid = lax.axis_index("c")
    sid = lax.axis_index("s")
    wid = sid * NC + cid
    base = wid * BPW

    def one(tab1, tab2, ids_hbm, out_hbm):
        pltpu.sync_copy(ids_hbm.at[pl.ds(base, BPW)], idx_v)
        cp1 = pltpu.async_copy(tab1.at[idx_v], a_v, sem)
        cp2 = pltpu.async_copy(tab2.at[idx_v], b_v, sem)
        cp1.wait()
        cp2.wait()

        def add_body(r, c):
            for cc in range(8):
                a_v[r, pl.ds(cc * 16, 16)] = (
                    a_v[r, pl.ds(cc * 16, 16)] + b_v[r, pl.ds(cc * 16, 16)])
            return c
        lax.fori_loop(0, BPW, add_body, 0)
        pltpu.sync_copy(a_v, out_hbm.at[pl.ds(base, BPW)])

    one(sgt_hbm, hgt_hbm, tid_hbm, selt_hbm)
    one(sgd_hbm, hgd_hbm, did_hbm, seld_hbm)


@functools.cache
def _select_call_build():
  return pl.kernel(
    _select_body,
    out_type=(jax.ShapeDtypeStruct((B, D), jnp.float32),
              jax.ShapeDtypeStruct((B, D), jnp.float32)),
    mesh=_sc_mesh(),
    scratch_types=(
        pltpu.VMEM((BPW,), jnp.int32),
        pltpu.VMEM((BPW, D), jnp.float32),
        pltpu.VMEM((BPW, D), jnp.float32),
        pltpu.SemaphoreType.DMA,
    ),
    compiler_params=pltpu.CompilerParams(needs_layout_passes=False),
  )


def _select_call(*args):
    return _select_call_build()(*args)


def _dotf(a, b):
    return jnp.dot(a, b, preferred_element_type=jnp.float32)


def _elu(x):
    return jnp.where(x > 0.0, x, jnp.exp(jnp.minimum(x, 0.0)) - 1.0)


def _gat_prep_kernel(xs_ref, xt_ref, ws_ref, wt_ref, a_ref,
                     s1_ref, s2_ref, hj_ref):
    a1 = a_ref[0:D, :]
    a2 = a_ref[D:2 * D, :]
    w1 = _dotf(ws_ref[...], a1)
    hj = _dotf(xt_ref[...], wt_ref[...])
    hj_ref[...] = hj
    s2_ref[...] = _dotf(hj, a2)
    s1_ref[...] = _dotf(xs_ref[...], w1)


def _gat_prep(x_src, x_tgt, p):
    return pl.pallas_call(
        _gat_prep_kernel,
        grid=(NBLK,),
        in_specs=[
            pl.BlockSpec((RB, D), lambda i: (i, 0)),
            pl.BlockSpec((RB, D), lambda i: (i, 0)),
            pl.BlockSpec((D, D), lambda i: (0, 0)),
            pl.BlockSpec((D, D), lambda i: (0, 0)),
            pl.BlockSpec((2 * D, 1), lambda i: (0, 0)),
        ],
        out_specs=[
            pl.BlockSpec((RB, 1), lambda i: (i, 0)),
            pl.BlockSpec((RB, 1), lambda i: (i, 0)),
            pl.BlockSpec((RB, D), lambda i: (i, 0)),
        ],
        out_shape=[
            jax.ShapeDtypeStruct((NP, 1), jnp.float32),
            jax.ShapeDtypeStruct((NP, 1), jnp.float32),
            jax.ShapeDtypeStruct((NP, D), jnp.float32),
        ],
    )(x_src, x_tgt, p['W_src'], p['W_tgt'], p['a'])


def _gat_combine_kernel(accr_ref, accd_ref, out_ref):
    s = accr_ref[0] + accr_ref[1]
    d = accd_ref[0] + accd_ref[1]
    out_ref[...] = _elu(s / (d + 1e-08))


def _gat_combine(accr, accd):
    return pl.pallas_call(
        _gat_combine_kernel,
        grid=(NBLK,),
        in_specs=[
            pl.BlockSpec((NC, RB, D), lambda i: (0, i, 0)),
            pl.BlockSpec((NC, RB, 1), lambda i: (0, i, 0)),
        ],
        out_specs=pl.BlockSpec((RB, D), lambda i: (i, 0)),
        out_shape=jax.ShapeDtypeStruct((NP, D), jnp.float32),
    )(accr, accd.reshape(NC, NP, 1))


def _hg_a_kernel(x_ref, hyper_ref, w_ref, s_ref, t_ref):
    i = pl.program_id(0)
    x = x_ref[...]
    s = _dotf(x, hyper_ref[...])
    s_ref[...] = s
    y = _dotf(x, w_ref[...])
    tt = lax.dot_general(s, y, (((0,), (0,)), ((), ())),
                         preferred_element_type=jnp.float32)

    @pl.when(i == 0)
    def _():
        t_ref[...] = tt

    @pl.when(i > 0)
    def _():
        t_ref[...] += tt


def _hg_b_kernel(s_ref, t_ref, out_ref):
    out_ref[...] = _elu(_dotf(s_ref[...], t_ref[...]))


def _hg(x, hyper, w):
    s, t = pl.pallas_call(
        _hg_a_kernel,
        grid=(NBLK,),
        in_specs=[
            pl.BlockSpec((RB, D), lambda i: (i, 0)),
            pl.BlockSpec((D, D), lambda i: (0, 0)),
            pl.BlockSpec((D, D), lambda i: (0, 0)),
        ],
        out_specs=[
            pl.BlockSpec((RB, D), lambda i: (i, 0)),
            pl.BlockSpec((D, D), lambda i: (0, 0)),
        ],
        out_shape=[
            jax.ShapeDtypeStruct((NP, D), jnp.float32),
            jax.ShapeDtypeStruct((D, D), jnp.float32),
        ],
        compiler_params=pltpu.CompilerParams(
            dimension_semantics=("arbitrary",)),
    )(x, hyper, w)
    return pl.pallas_call(
        _hg_b_kernel,
        grid=(NBLK,),
        in_specs=[
            pl.BlockSpec((RB, D), lambda i: (i, 0)),
            pl.BlockSpec((D, D), lambda i: (0, 0)),
        ],
        out_specs=pl.BlockSpec((RB, D), lambda i: (i, 0)),
        out_shape=jax.ShapeDtypeStruct((NP, D), jnp.float32),
    )(s, t)


def _norm_kernel(z1_ref, z2_ref, z1n_ref, z2n_ref, pos_ref):
    z1 = z1_ref[...]
    z2 = z2_ref[...]
    n1 = jnp.sqrt(jnp.sum(z1 * z1, axis=1, keepdims=True))
    n2 = jnp.sqrt(jnp.sum(z2 * z2, axis=1, keepdims=True))
    z1n = z1 / (n1 + 1e-12)
    z2n = z2 / (n2 + 1e-12)
    z1n_ref[...] = z1n
    z2n_ref[...] = z2n
    pos_ref[...] = jnp.sum(z1n * z2n, axis=1, keepdims=True) / TEMP


def _nce_kernel(z1n_ref, z2n_ref, pos_ref, out_ref, acc_ref):
    r = pl.program_id(0)
    c = pl.program_id(1)
    sim = lax.dot_general(z1n_ref[...], z2n_ref[...],
                          (((1,), (1,)), ((), ())),
                          preferred_element_type=jnp.float32) / TEMP
    e = jnp.exp(sim)
    colid = c * RB + lax.broadcasted_iota(jnp.int32, (RB, RB), 1)
    e = jnp.where(colid < N, e, 0.0)
    rowsum = jnp.sum(e, axis=1, keepdims=True)

    @pl.when(c == 0)
    def _():
        acc_ref[...] = rowsum

    @pl.when(c > 0)
    def _():
        acc_ref[...] += rowsum

    @pl.when(c == NBLK - 1)
    def _():
        lse = jnp.log(acc_ref[...])
        contrib = lse - pos_ref[...]
        rowid = r * RB + lax.broadcasted_iota(jnp.int32, (RB, 1), 0)
        contrib = jnp.where(rowid < N, contrib, 0.0)
        out_ref[...] = jnp.sum(contrib).reshape(1, 1, 1)


def _info_nce(z1, z2):
    z1n, z2n, pos = pl.pallas_call(
        _norm_kernel,
        grid=(NBLK,),
        in_specs=[
            pl.BlockSpec((RB, D), lambda i: (i, 0)),
            pl.BlockSpec((RB, D), lambda i: (i, 0)),
        ],
        out_specs=[
            pl.BlockSpec((RB, D), lambda i: (i, 0)),
            pl.BlockSpec((RB, D), lambda i: (i, 0)),
            pl.BlockSpec((RB, 1), lambda i: (i, 0)),
        ],
        out_shape=[
            jax.ShapeDtypeStruct((NP, D), jnp.float32),
            jax.ShapeDtypeStruct((NP, D), jnp.float32),
            jax.ShapeDtypeStruct((NP, 1), jnp.float32),
        ],
    )(z1, z2)
    parts = pl.pallas_call(
        _nce_kernel,
        grid=(NBLK, NBLK),
        in_specs=[
            pl.BlockSpec((RB, D), lambda r, c: (r, 0)),
            pl.BlockSpec((RB, D), lambda r, c: (c, 0)),
            pl.BlockSpec((RB, 1), lambda r, c: (r, 0)),
        ],
        out_specs=pl.BlockSpec((1, 1, 1), lambda r, c: (r, 0, 0)),
        out_shape=jax.ShapeDtypeStruct((NBLK, 1, 1), jnp.float32),
        scratch_shapes=[pltpu.VMEM((RB, 1), jnp.float32)],
        compiler_params=pltpu.CompilerParams(
            dimension_semantics=("arbitrary", "arbitrary")),
    )(z1n, z2n, pos)
    return jnp.sum(parts) / N


def _head_kernel(st_ref, sd_ref, w1_ref, b1_ref, w2_ref, b2_ref,
                 gw_ref, gb_ref, fus_ref, gmf_ref, mlp_ref):
    st = st_ref[...]
    sd = sd_ref[...]
    gmf_vec = st * sd
    gmf = jnp.sum(gmf_vec, axis=1, keepdims=True)
    hid = _dotf(st, w1_ref[0:D, :]) + _dotf(sd, w1_ref[D:2 * D, :])
    hid = jnp.maximum(hid + b1_ref[...], 0.0)
    mlp = _dotf(hid, w2_ref[...]) + b2_ref[...]
    gate = 1.0 / (1.0 + jnp.exp(-(_dotf(gmf_vec, gw_ref[...]) + gb_ref[...])))
    fus_ref[...] = gate * gmf + (1.0 - gate) * mlp
    gmf_ref[...] = gmf
    mlp_ref[...] = mlp


def _head(sel_t, sel_d, p):
    hb = B // 4
    return pl.pallas_call(
        _head_kernel,
        grid=(4,),
        in_specs=[
            pl.BlockSpec((hb, D), lambda i: (i, 0)),
            pl.BlockSpec((hb, D), lambda i: (i, 0)),
            pl.BlockSpec((2 * D, D), lambda i: (0, 0)),
            pl.BlockSpec((1, D), lambda i: (0, 0)),
            pl.BlockSpec((D, 1), lambda i: (0, 0)),
            pl.BlockSpec((1, 1), lambda i: (0, 0)),
            pl.BlockSpec((D, 1), lambda i: (0, 0)),
            pl.BlockSpec((1, 1), lambda i: (0, 0)),
        ],
        out_specs=[
            pl.BlockSpec((hb, 1), lambda i: (i, 0)),
            pl.BlockSpec((hb, 1), lambda i: (i, 0)),
            pl.BlockSpec((hb, 1), lambda i: (i, 0)),
        ],
        out_shape=[
            jax.ShapeDtypeStruct((B, 1), jnp.float32),
            jax.ShapeDtypeStruct((B, 1), jnp.float32),
            jax.ShapeDtypeStruct((B, 1), jnp.float32),
        ],
    )(sel_t, sel_d, p['mlp_W1'], p['mlp_b1'].reshape(1, D),
      p['mlp_W2'], p['mlp_b2'].reshape(1, 1),
      p['gate_W'], p['gate_b'].reshape(1, 1))


def _pad_rows(x):
    return jnp.pad(x, ((0, NP - N), (0, 0)))


def _gat(x_src, x_tgt, src2, tgt2, p, zr, zd):
    s1, s2, hj = _gat_prep(x_src, x_tgt, p)
    accr, accd = _edge_call(src2, tgt2, s1.reshape(NP), s2.reshape(NP),
                            hj, zr, zd)
    return _gat_combine(accr, accd)


def kernel(params, adj_src, adj_dst, trna_ids, disease_ids):
    p = params
    src2 = adj_src.astype(jnp.int32).reshape(E // 128, 128)
    dst2 = adj_dst.astype(jnp.int32).reshape(E // 128, 128)
    zr = jnp.zeros((SLAB, D), jnp.float32)
    zd = jnp.zeros((SLAB,), jnp.float32)

    sg_t = _pad_rows(p['trna_embed'])
    sg_d = _pad_rows(p['disease_embed'])

    sg_t = _gat(sg_t, sg_d, src2, dst2, p['gat_t'][0], zr, zd)
    sg_d = _gat(sg_d, sg_t, dst2, src2, p['gat_d'][0], zr, zd)

    hg_t = _hg(sg_t, p['thyper'], p['hg_t'][0])
    hg_d = _hg(sg_d, p['dhyper'], p['hg_d'][0])
    hg_t = _hg(hg_t, p['thyper'], p['hg_t'][1])
    hg_d = _hg(hg_d, p['dhyper'], p['hg_d'][1])

    sg_t = _gat(hg_t, hg_d, src2, dst2, p['gat_t'][1], zr, zd)
    sg_d = _gat(hg_d, hg_t, dst2, src2, p['gat_d'][1], zr, zd)
    sg_t = _gat(sg_t, sg_d, src2, dst2, p['gat_t'][2], zr, zd)
    sg_d = _gat(sg_d, sg_t, dst2, src2, p['gat_d'][2], zr, zd)

    hg_t2 = _hg(hg_t, p['thyper'], p['hg_t'][2])
    hg_d2 = _hg(hg_d, p['dhyper'], p['hg_d'][2])

    contrast = _info_nce(sg_t, hg_t2) + _info_nce(sg_d, hg_d2)

    sel_t, sel_d = _select_call(sg_t, hg_t2, sg_d, hg_d2,
                                trna_ids.astype(jnp.int32),
                                disease_ids.astype(jnp.int32))
    fus, gmf, mlp = _head(sel_t, sel_d, p)

    return (fus[:, 0], sg_t[:N], sg_d[:N], gmf[:, 0], mlp[:, 0], contrast)


# speedup vs baseline: 12.2304x; 1.5516x over previous
import functools

import jax
import jax.numpy as jnp
from jax import lax
from jax.experimental import pallas as pl
from jax.experimental.pallas import tpu as pltpu
from jax.experimental.pallas import tpu_sc as plsc

N = 10000
D = 128
E = 320000
B = 4096
LEAK = 0.2
TEMP = 0.5

NP = 10240
RB = 1024
NBLK = NP // RB

NC = 2
NS = 16
NW = NC * NS
CHUNK = 80
EPW = E // NW
WCH = EPW // CHUNK
IDXCOLS = CHUNK
SLAB = NP // NS

def _sc_mesh():
    return plsc.VectorSubcoreMesh(core_axis_name="c", subcore_axis_name="s",
                                  num_cores=NC, num_subcores=NS)


def _edge_body(src2_hbm, tgt2_hbm, s1_hbm, s2_hbm, hj_hbm, zr_hbm, zd_hbm,
               accr_hbm, accd_hbm,
               srcall_v, tgtall_v, rows0, rows1, sv1_0, sv1_1,
               sv2_0, sv2_1, w0, w1, gsem0, gsem1,
               sh_rows, sh_den):
    cid = lax.axis_index("c")
    sid = lax.axis_index("s")
    wid = sid * NC + cid

    bufs = ((rows0, sv1_0, sv2_0, w0, gsem0),
            (rows1, sv1_1, sv2_1, w1, gsem1))

    pltpu.sync_copy(src2_hbm.at[pl.ds(wid * WCH, WCH)], srcall_v)
    pltpu.sync_copy(tgt2_hbm.at[pl.ds(wid * WCH, WCH)], tgtall_v)
    base = sid * SLAB
    pltpu.sync_copy(zr_hbm, sh_rows.at[pl.ds(base, SLAB)])
    pltpu.sync_copy(zd_hbm, sh_den.at[pl.ds(base, SLAB)])
    plsc.subcore_barrier()

    def fire(c, buf):
        rows, sv1, sv2, _, sem = buf
        pltpu.async_copy(hj_hbm.at[tgtall_v.at[c]], rows, sem)
        pltpu.async_copy(s1_hbm.at[srcall_v.at[c]], sv1, sem)
        pltpu.async_copy(s2_hbm.at[tgtall_v.at[c]], sv2, sem)

    def drain(buf):
        rows, sv1, sv2, _, sem = buf
        pltpu.make_async_copy(hj_hbm.at[tgtall_v.at[0]], rows, sem).wait()
        pltpu.make_async_copy(s1_hbm.at[srcall_v.at[0]], sv1, sem).wait()
        pltpu.make_async_copy(s2_hbm.at[tgtall_v.at[0]], sv2, sem).wait()

    def process(c, buf):
        rows, sv1, sv2, w, _ = buf
        drain(buf)
        for g in range(CHUNK // 16):
            e = sv1[pl.ds(g * 16, 16)] + sv2[pl.ds(g * 16, 16)]
            e = jnp.where(e >= 0.0, e, e * LEAK)
            e = jnp.clip(e, -30.0, 30.0)
            w[pl.ds(g * 16, 16)] = jnp.exp(e)

        def scale_body(r, cc):
            wspl = plsc.load_gather(w, [jnp.full((16,), 0, jnp.int32) + r])
            for k in range(8):
                rows[r, pl.ds(k * 16, 16)] = rows[r, pl.ds(k * 16, 16)] * wspl
            return cc
        lax.fori_loop(0, CHUNK, scale_body, 0)

        pltpu.sync_copy(rows, sh_rows.at[srcall_v.at[c]], add=True)
        pltpu.sync_copy(w, sh_den.at[srcall_v.at[c]], add=True)

    fire(0, bufs[0])

    def pair_body(jp, carry):
        c0 = 2 * jp
        fire(c0 + 1, bufs[1])
        process(c0, bufs[0])
        fire(c0 + 2, bufs[0])
        process(c0 + 1, bufs[1])
        return carry

    lax.fori_loop(0, (WCH - 1) // 2, pair_body, 0)
    process(WCH - 1, bufs[0])

    plsc.subcore_barrier()
    pltpu.sync_copy(sh_rows.at[pl.ds(base, SLAB)],
                    accr_hbm.at[cid, pl.ds(base, SLAB)])
    pltpu.sync_copy(sh_den.at[pl.ds(base, SLAB)],
                    accd_hbm.at[cid, pl.ds(base, SLAB)])


@functools.cache
def _edge_call_build():
  return pl.kernel(
    _edge_body,
    out_type=(jax.ShapeDtypeStruct((NC, NP, D), jnp.float32),
              jax.ShapeDtypeStruct((NC, NP), jnp.float32)),
    mesh=_sc_mesh(),
    scratch_types=(
        pltpu.VMEM((WCH, CHUNK), jnp.int32),
        pltpu.VMEM((WCH, CHUNK), jnp.int32),
        pltpu.VMEM((CHUNK, D), jnp.float32),
        pltpu.VMEM((CHUNK, D), jnp.float32),
        pltpu.VMEM((CHUNK,), jnp.float32),
        pltpu.VMEM((CHUNK,), jnp.float32),
        pltpu.VMEM((CHUNK,), jnp.float32),
        pltpu.VMEM((CHUNK,), jnp.float32),
        pltpu.VMEM((CHUNK,), jnp.float32),
        pltpu.VMEM((CHUNK,), jnp.float32),
        pltpu.SemaphoreType.DMA,
        pltpu.SemaphoreType.DMA,
        pltpu.VMEM_SHARED((NP, D), jnp.float32),
        pltpu.VMEM_SHARED((NP,), jnp.float32),
    ),
    compiler_params=pltpu.CompilerParams(needs_layout_passes=False,
                                         use_tc_tiling_on_sc=False),
  )


def _edge_call(*args):
    return _edge_call_build()(*args)


BPW = B // NW


def _select_body(sgt_hbm, hgt_hbm, sgd_hbm, hgd_hbm, tid_hbm, did_hbm,
                 selt_hbm, seld_hbm,
                 idx_v, a_v, b_v, sem):
    cid = lax.axis_index("c")
    sid = lax.axis_index("s")
    wid = sid * NC + cid
    base = wid * BPW

    def one(tab1, tab2, ids_hbm, out_hbm):
        pltpu.sync_copy(ids_hbm.at[pl.ds(base, BPW)], idx_v)
        cp1 = pltpu.async_copy(tab1.at[idx_v], a_v, sem)
        cp2 = pltpu.async_copy(tab2.at[idx_v], b_v, sem)
        cp1.wait()
        cp2.wait()

        def add_body(r, c):
            for cc in range(8):
                a_v[r, pl.ds(cc * 16, 16)] = (
                    a_v[r, pl.ds(cc * 16, 16)] + b_v[r, pl.ds(cc * 16, 16)])
            return c
        lax.fori_loop(0, BPW, add_body, 0)
        pltpu.sync_copy(a_v, out_hbm.at[pl.ds(base, BPW)])

    one(sgt_hbm, hgt_hbm, tid_hbm, selt_hbm)
    one(sgd_hbm, hgd_hbm, did_hbm, seld_hbm)


@functools.cache
def _select_call_build():
  return pl.kernel(
    _select_body,
    out_type=(jax.ShapeDtypeStruct((B, D), jnp.float32),
              jax.ShapeDtypeStruct((B, D), jnp.float32)),
    mesh=_sc_mesh(),
    scratch_types=(
        pltpu.VMEM((BPW,), jnp.int32),
        pltpu.VMEM((BPW, D), jnp.float32),
        pltpu.VMEM((BPW, D), jnp.float32),
        pltpu.SemaphoreType.DMA,
    ),
    compiler_params=pltpu.CompilerParams(needs_layout_passes=False),
  )


def _select_call(*args):
    return _select_call_build()(*args)


def _dotf(a, b):
    return jnp.dot(a, b, preferred_element_type=jnp.float32)


def _elu(x):
    return jnp.where(x > 0.0, x, jnp.exp(jnp.minimum(x, 0.0)) - 1.0)


def _gat_prep_kernel(xs_ref, xt_ref, ws_ref, wt_ref, a_ref,
                     s1_ref, s2_ref, hj_ref):
    a1 = a_ref[0:D, :]
    a2 = a_ref[D:2 * D, :]
    w1 = _dotf(ws_ref[...], a1)
    hj = _dotf(xt_ref[...], wt_ref[...])
    hj_ref[...] = hj
    s2_ref[...] = _dotf(hj, a2)
    s1_ref[...] = _dotf(xs_ref[...], w1)


def _gat_prep(x_src, x_tgt, p):
    return pl.pallas_call(
        _gat_prep_kernel,
        grid=(NBLK,),
        in_specs=[
            pl.BlockSpec((RB, D), lambda i: (i, 0)),
            pl.BlockSpec((RB, D), lambda i: (i, 0)),
            pl.BlockSpec((D, D), lambda i: (0, 0)),
            pl.BlockSpec((D, D), lambda i: (0, 0)),
            pl.BlockSpec((2 * D, 1), lambda i: (0, 0)),
        ],
        out_specs=[
            pl.BlockSpec((RB, 1), lambda i: (i, 0)),
            pl.BlockSpec((RB, 1), lambda i: (i, 0)),
            pl.BlockSpec((RB, D), lambda i: (i, 0)),
        ],
        out_shape=[
            jax.ShapeDtypeStruct((NP, 1), jnp.float32),
            jax.ShapeDtypeStruct((NP, 1), jnp.float32),
            jax.ShapeDtypeStruct((NP, D), jnp.float32),
        ],
    )(x_src, x_tgt, p['W_src'], p['W_tgt'], p['a'])


def _gat_combine_kernel(accr_ref, accd_ref, out_ref):
    s = accr_ref[0] + accr_ref[1]
    d = accd_ref[0] + accd_ref[1]
    out_ref[...] = _elu(s / (d + 1e-08))


def _gat_combine(accr, accd):
    return pl.pallas_call(
        _gat_combine_kernel,
        grid=(NBLK,),
        in_specs=[
            pl.BlockSpec((NC, RB, D), lambda i: (0, i, 0)),
            pl.BlockSpec((NC, RB, 1), lambda i: (0, i, 0)),
        ],
        out_specs=pl.BlockSpec((RB, D), lambda i: (i, 0)),
        out_shape=jax.ShapeDtypeStruct((NP, D), jnp.float32),
    )(accr, accd.reshape(NC, NP, 1))


def _hg_a_kernel(x_ref, hyper_ref, w_ref, s_ref, t_ref):
    i = pl.program_id(0)
    x = x_ref[...]
    s = _dotf(x, hyper_ref[...])
    s_ref[...] = s
    y = _dotf(x, w_ref[...])
    tt = lax.dot_general(s, y, (((0,), (0,)), ((), ())),
                         preferred_element_type=jnp.float32)

    @pl.when(i == 0)
    def _():
        t_ref[...] = tt

    @pl.when(i > 0)
    def _():
        t_ref[...] += tt


def _hg_b_kernel(s_ref, t_ref, out_ref):
    out_ref[...] = _elu(_dotf(s_ref[...], t_ref[...]))


def _hg(x, hyper, w):
    s, t = pl.pallas_call(
        _hg_a_kernel,
        grid=(NBLK,),
        in_specs=[
            pl.BlockSpec((RB, D), lambda i: (i, 0)),
            pl.BlockSpec((D, D), lambda i: (0, 0)),
            pl.BlockSpec((D, D), lambda i: (0, 0)),
        ],
        out_specs=[
            pl.BlockSpec((RB, D), lambda i: (i, 0)),
            pl.BlockSpec((D, D), lambda i: (0, 0)),
        ],
        out_shape=[
            jax.ShapeDtypeStruct((NP, D), jnp.float32),
            jax.ShapeDtypeStruct((D, D), jnp.float32),
        ],
        compiler_params=pltpu.CompilerParams(
            dimension_semantics=("arbitrary",)),
    )(x, hyper, w)
    return pl.pallas_call(
        _hg_b_kernel,
        grid=(NBLK,),
        in_specs=[
            pl.BlockSpec((RB, D), lambda i: (i, 0)),
            pl.BlockSpec((D, D), lambda i: (0, 0)),
        ],
        out_specs=pl.BlockSpec((RB, D), lambda i: (i, 0)),
        out_shape=jax.ShapeDtypeStruct((NP, D), jnp.float32),
    )(s, t)


def _norm_kernel(z1_ref, z2_ref, z1n_ref, z2n_ref, pos_ref):
    z1 = z1_ref[...]
    z2 = z2_ref[...]
    n1 = jnp.sqrt(jnp.sum(z1 * z1, axis=1, keepdims=True))
    n2 = jnp.sqrt(jnp.sum(z2 * z2, axis=1, keepdims=True))
    z1n = z1 / (n1 + 1e-12)
    z2n = z2 / (n2 + 1e-12)
    z1n_ref[...] = z1n
    z2n_ref[...] = z2n
    pos_ref[...] = jnp.sum(z1n * z2n, axis=1, keepdims=True) / TEMP


def _nce_kernel(z1n_ref, z2n_ref, pos_ref, out_ref, acc_ref):
    r = pl.program_id(0)
    c = pl.program_id(1)
    sim = lax.dot_general(z1n_ref[...], z2n_ref[...],
                          (((1,), (1,)), ((), ())),
                          preferred_element_type=jnp.float32) / TEMP
    e = jnp.exp(sim)
    colid = c * RB + lax.broadcasted_iota(jnp.int32, (RB, RB), 1)
    e = jnp.where(colid < N, e, 0.0)
    rowsum = jnp.sum(e, axis=1, keepdims=True)

    @pl.when(c == 0)
    def _():
        acc_ref[...] = rowsum

    @pl.when(c > 0)
    def _():
        acc_ref[...] += rowsum

    @pl.when(c == NBLK - 1)
    def _():
        lse = jnp.log(acc_ref[...])
        contrib = lse - pos_ref[...]
        rowid = r * RB + lax.broadcasted_iota(jnp.int32, (RB, 1), 0)
        contrib = jnp.where(rowid < N, contrib, 0.0)
        out_ref[...] = jnp.sum(contrib).reshape(1, 1, 1)


def _info_nce(z1, z2):
    z1n, z2n, pos = pl.pallas_call(
        _norm_kernel,
        grid=(NBLK,),
        in_specs=[
            pl.BlockSpec((RB, D), lambda i: (i, 0)),
            pl.BlockSpec((RB, D), lambda i: (i, 0)),
        ],
        out_specs=[
            pl.BlockSpec((RB, D), lambda i: (i, 0)),
            pl.BlockSpec((RB, D), lambda i: (i, 0)),
            pl.BlockSpec((RB, 1), lambda i: (i, 0)),
        ],
        out_shape=[
            jax.ShapeDtypeStruct((NP, D), jnp.float32),
            jax.ShapeDtypeStruct((NP, D), jnp.float32),
            jax.ShapeDtypeStruct((NP, 1), jnp.float32),
        ],
    )(z1, z2)
    parts = pl.pallas_call(
        _nce_kernel,
        grid=(NBLK, NBLK),
        in_specs=[
            pl.BlockSpec((RB, D), lambda r, c: (r, 0)),
            pl.BlockSpec((RB, D), lambda r, c: (c, 0)),
            pl.BlockSpec((RB, 1), lambda r, c: (r, 0)),
        ],
        out_specs=pl.BlockSpec((1, 1, 1), lambda r, c: (r, 0, 0)),
        out_shape=jax.ShapeDtypeStruct((NBLK, 1, 1), jnp.float32),
        scratch_shapes=[pltpu.VMEM((RB, 1), jnp.float32)],
        compiler_params=pltpu.CompilerParams(
            dimension_semantics=("arbitrary", "arbitrary")),
    )(z1n, z2n, pos)
    return jnp.sum(parts) / N


def _head_kernel(st_ref, sd_ref, w1_ref, b1_ref, w2_ref, b2_ref,
                 gw_ref, gb_ref, fus_ref, gmf_ref, mlp_ref):
    st = st_ref[...]
    sd = sd_ref[...]
    gmf_vec = st * sd
    gmf = jnp.sum(gmf_vec, axis=1, keepdims=True)
    hid = _dotf(st, w1_ref[0:D, :]) + _dotf(sd, w1_ref[D:2 * D, :])
    hid = jnp.maximum(hid + b1_ref[...], 0.0)
    mlp = _dotf(hid, w2_ref[...]) + b2_ref[...]
    gate = 1.0 / (1.0 + jnp.exp(-(_dotf(gmf_vec, gw_ref[...]) + gb_ref[...])))
    fus_ref[...] = gate * gmf + (1.0 - gate) * mlp
    gmf_ref[...] = gmf
    mlp_ref[...] = mlp


def _head(sel_t, sel_d, p):
    hb = B // 4
    return pl.pallas_call(
        _head_kernel,
        grid=(4,),
        in_specs=[
            pl.BlockSpec((hb, D), lambda i: (i, 0)),
            pl.BlockSpec((hb, D), lambda i: (i, 0)),
            pl.BlockSpec((2 * D, D), lambda i: (0, 0)),
            pl.BlockSpec((1, D), lambda i: (0, 0)),
            pl.BlockSpec((D, 1), lambda i: (0, 0)),
            pl.BlockSpec((1, 1), lambda i: (0, 0)),
            pl.BlockSpec((D, 1), lambda i: (0, 0)),
            pl.BlockSpec((1, 1), lambda i: (0, 0)),
        ],
        out_specs=[
            pl.BlockSpec((hb, 1), lambda i: (i, 0)),
            pl.BlockSpec((hb, 1), lambda i: (i, 0)),
            pl.BlockSpec((hb, 1), lambda i: (i, 0)),
        ],
        out_shape=[
            jax.ShapeDtypeStruct((B, 1), jnp.float32),
            jax.ShapeDtypeStruct((B, 1), jnp.float32),
            jax.ShapeDtypeStruct((B, 1), jnp.float32),
        ],
    )(sel_t, sel_d, p['mlp_W1'], p['mlp_b1'].reshape(1, D),
      p['mlp_W2'], p['mlp_b2'].reshape(1, 1),
      p['gate_W'], p['gate_b'].reshape(1, 1))


def _pad_rows(x):
    return jnp.pad(x, ((0, NP - N), (0, 0)))


def _gat(x_src, x_tgt, src2, tgt2, p, zr, zd):
    s1, s2, hj = _gat_prep(x_src, x_tgt, p)
    accr, accd = _edge_call(src2, tgt2, s1.reshape(NP), s2.reshape(NP),
                            hj, zr, zd)
    return _gat_combine(accr, accd)


def kernel(params, adj_src, adj_dst, trna_ids, disease_ids):
    p = params
    src2 = adj_src.astype(jnp.int32).reshape(E // CHUNK, CHUNK)
    dst2 = adj_dst.astype(jnp.int32).reshape(E // CHUNK, CHUNK)
    zr = jnp.zeros((SLAB, D), jnp.float32)
    zd = jnp.zeros((SLAB,), jnp.float32)

    sg_t = _pad_rows(p['trna_embed'])
    sg_d = _pad_rows(p['disease_embed'])

    sg_t = _gat(sg_t, sg_d, src2, dst2, p['gat_t'][0], zr, zd)
    sg_d = _gat(sg_d, sg_t, dst2, src2, p['gat_d'][0], zr, zd)

    hg_t = _hg(sg_t, p['thyper'], p['hg_t'][0])
    hg_d = _hg(sg_d, p['dhyper'], p['hg_d'][0])
    hg_t = _hg(hg_t, p['thyper'], p['hg_t'][1])
    hg_d = _hg(hg_d, p['dhyper'], p['hg_d'][1])

    sg_t = _gat(hg_t, hg_d, src2, dst2, p['gat_t'][1], zr, zd)
    sg_d = _gat(hg_d, hg_t, dst2, src2, p['gat_d'][1], zr, zd)
    sg_t = _gat(sg_t, sg_d, src2, dst2, p['gat_t'][2], zr, zd)
    sg_d = _gat(sg_d, sg_t, dst2, src2, p['gat_d'][2], zr, zd)

    hg_t2 = _hg(hg_t, p['thyper'], p['hg_t'][2])
    hg_d2 = _hg(hg_d, p['dhyper'], p['hg_d'][2])

    contrast = _info_nce(sg_t, hg_t2) + _info_nce(sg_d, hg_d2)

    sel_t, sel_d = _select_call(sg_t, hg_t2, sg_d, hg_d2,
                                trna_ids.astype(jnp.int32),
                                disease_ids.astype(jnp.int32))
    fus, gmf, mlp = _head(sel_t, sel_d, p)

    return (fus[:, 0], sg_t[:N], sg_d[:N], gmf[:, 0], mlp[:, 0], contrast)


# speedup vs baseline: 13.8244x; 1.1303x over previous
import functools

import jax
import jax.numpy as jnp
from jax import lax
from jax.experimental import pallas as pl
from jax.experimental.pallas import tpu as pltpu
from jax.experimental.pallas import tpu_sc as plsc

N = 10000
D = 128
E = 320000
B = 4096
LEAK = 0.2
TEMP = 0.5

NP = 10240
RB = 1024
NBLK = NP // RB

NC = 2
NS = 16
NW = NC * NS
CHUNK = 80
EPW = E // NW
WCH = EPW // CHUNK
IDXCOLS = CHUNK
SLAB = NP // NS

def _sc_mesh():
    return plsc.VectorSubcoreMesh(core_axis_name="c", subcore_axis_name="s",
                                  num_cores=NC, num_subcores=NS)


def _edge_body(pk_hbm, s1_hbm, s2_hbm, hj_hbm, zr_hbm, zd_hbm,
               accr_hbm, accd_hbm,
               pkall_v,
               rows0, rows1, rows2, sv1_0, sv1_1, sv1_2,
               sv2_0, sv2_1, sv2_2, w0, w1, w2,
               si0, si1, si2, ti0, ti1, ti2,
               gsem0, gsem1, gsem2, ssem0, ssem1, ssem2,
               sh_rows, sh_den):
    cid = lax.axis_index("c")
    sid = lax.axis_index("s")
    wid = sid * NC + cid

    bufs = ((rows0, sv1_0, sv2_0, w0, si0, ti0, gsem0, ssem0),
            (rows1, sv1_1, sv2_1, w1, si1, ti1, gsem1, ssem1),
            (rows2, sv1_2, sv2_2, w2, si2, ti2, gsem2, ssem2))

    pltpu.sync_copy(pk_hbm.at[pl.ds(wid * WCH, WCH)], pkall_v)
    base = sid * SLAB
    pltpu.sync_copy(zr_hbm, sh_rows.at[pl.ds(base, SLAB)])
    pltpu.sync_copy(zd_hbm, sh_den.at[pl.ds(base, SLAB)])
    plsc.subcore_barrier()

    def fire(c, buf):
        rows, sv1, sv2, _, si, ti, gsem, _ = buf
        for g in range(CHUNK // 16):
            pk = pkall_v[c, pl.ds(g * 16, 16)]
            si[pl.ds(g * 16, 16)] = lax.shift_right_logical(pk, 16)
            ti[pl.ds(g * 16, 16)] = lax.bitwise_and(pk, 0xFFFF)
        pltpu.async_copy(hj_hbm.at[ti], rows, gsem)
        pltpu.async_copy(s1_hbm.at[si], sv1, gsem)
        pltpu.async_copy(s2_hbm.at[ti], sv2, gsem)

    def drain_sc(buf):
        rows, _, _, w, si, _, _, ssem = buf
        pltpu.make_async_copy(rows, sh_rows.at[si], ssem).wait()
        pltpu.make_async_copy(w, sh_den.at[si], ssem).wait()

    def process(c, buf):
        rows, sv1, sv2, w, si, ti, gsem, ssem = buf
        pltpu.make_async_copy(hj_hbm.at[ti], rows, gsem).wait()
        pltpu.make_async_copy(s1_hbm.at[si], sv1, gsem).wait()
        pltpu.make_async_copy(s2_hbm.at[ti], sv2, gsem).wait()
        for g in range(CHUNK // 16):
            e = sv1[pl.ds(g * 16, 16)] + sv2[pl.ds(g * 16, 16)]
            e = jnp.where(e >= 0.0, e, e * LEAK)
            e = jnp.clip(e, -30.0, 30.0)
            w[pl.ds(g * 16, 16)] = jnp.exp(e)

        def scale_body(r, cc):
            wspl = plsc.load_gather(w, [jnp.full((16,), 0, jnp.int32) + r])
            for k in range(8):
                rows[r, pl.ds(k * 16, 16)] = rows[r, pl.ds(k * 16, 16)] * wspl
            return cc
        lax.fori_loop(0, CHUNK, scale_body, 0, unroll=2)

        pltpu.async_copy(rows, sh_rows.at[si], ssem, add=True)
        pltpu.async_copy(w, sh_den.at[si], ssem, add=True)

    def step(c, cur, nxt2):
        process(c, cur)
        drain_sc(nxt2)
        fire(c + 2, nxt2)

    fire(0, bufs[0])
    fire(1, bufs[1])
    process(0, bufs[0])
    fire(2, bufs[2])
    process(1, bufs[1])
    drain_sc(bufs[0])
    fire(3, bufs[0])
    process(2, bufs[2])
    drain_sc(bufs[1])
    fire(4, bufs[1])

    def loop_body(jp, carry):
        c = 3 * jp + 3
        step(c, bufs[0], bufs[2])
        step(c + 1, bufs[1], bufs[0])
        step(c + 2, bufs[2], bufs[1])
        return carry

    lax.fori_loop(0, (WCH - 5) // 3, loop_body, 0)
    process(WCH - 2, bufs[0])
    drain_sc(bufs[2])
    process(WCH - 1, bufs[1])
    drain_sc(bufs[0])
    drain_sc(bufs[1])

    plsc.subcore_barrier()
    pltpu.sync_copy(sh_rows.at[pl.ds(base, SLAB)],
                    accr_hbm.at[cid, pl.ds(base, SLAB)])
    pltpu.sync_copy(sh_den.at[pl.ds(base, SLAB)],
                    accd_hbm.at[cid, pl.ds(base, SLAB)])


@functools.cache
def _edge_call_build():
  return pl.kernel(
    _edge_body,
    out_type=(jax.ShapeDtypeStruct((NC, NP, D), jnp.float32),
              jax.ShapeDtypeStruct((NC, NP), jnp.float32)),
    mesh=_sc_mesh(),
    scratch_types=(
        pltpu.VMEM((WCH, CHUNK), jnp.int32),
        pltpu.VMEM((CHUNK, D), jnp.float32),
        pltpu.VMEM((CHUNK, D), jnp.float32),
        pltpu.VMEM((CHUNK, D), jnp.float32),
        pltpu.VMEM((CHUNK,), jnp.float32),
        pltpu.VMEM((CHUNK,), jnp.float32),
        pltpu.VMEM((CHUNK,), jnp.float32),
        pltpu.VMEM((CHUNK,), jnp.float32),
        pltpu.VMEM((CHUNK,), jnp.float32),
        pltpu.VMEM((CHUNK,), jnp.float32),
        pltpu.VMEM((CHUNK,), jnp.float32),
        pltpu.VMEM((CHUNK,), jnp.float32),
        pltpu.VMEM((CHUNK,), jnp.float32),
        pltpu.VMEM((CHUNK,), jnp.int32),
        pltpu.VMEM((CHUNK,), jnp.int32),
        pltpu.VMEM((CHUNK,), jnp.int32),
        pltpu.VMEM((CHUNK,), jnp.int32),
        pltpu.VMEM((CHUNK,), jnp.int32),
        pltpu.VMEM((CHUNK,), jnp.int32),
        pltpu.SemaphoreType.DMA,
        pltpu.SemaphoreType.DMA,
        pltpu.SemaphoreType.DMA,
        pltpu.SemaphoreType.DMA,
        pltpu.SemaphoreType.DMA,
        pltpu.SemaphoreType.DMA,
        pltpu.VMEM_SHARED((NP, D), jnp.float32),
        pltpu.VMEM_SHARED((NP,), jnp.float32),
    ),
    compiler_params=pltpu.CompilerParams(needs_layout_passes=False,
                                         use_tc_tiling_on_sc=False),
  )


def _edge_call(*args):
    return _edge_call_build()(*args)


BPW = B // NW


def _select_body(sgt_hbm, hgt_hbm, sgd_hbm, hgd_hbm, tid_hbm, did_hbm,
                 selt_hbm, seld_hbm,
                 idx_v, a_v, b_v, sem):
    cid = lax.axis_index("c")
    sid = lax.axis_index("s")
    wid = sid * NC + cid
    base = wid * BPW

    def one(tab1, tab2, ids_hbm, out_hbm):
        pltpu.sync_copy(ids_hbm.at[pl.ds(base, BPW)], idx_v)
        cp1 = pltpu.async_copy(tab1.at[idx_v], a_v, sem)
        cp2 = pltpu.async_copy(tab2.at[idx_v], b_v, sem)
        cp1.wait()
        cp2.wait()

        def add_body(r, c):
            for cc in range(8):
                a_v[r, pl.ds(cc * 16, 16)] = (
                    a_v[r, pl.ds(cc * 16, 16)] + b_v[r, pl.ds(cc * 16, 16)])
            return c
        lax.fori_loop(0, BPW, add_body, 0)
        pltpu.sync_copy(a_v, out_hbm.at[pl.ds(base, BPW)])

    one(sgt_hbm, hgt_hbm, tid_hbm, selt_hbm)
    one(sgd_hbm, hgd_hbm, did_hbm, seld_hbm)


@functools.cache
def _select_call_build():
  return pl.kernel(
    _select_body,
    out_type=(jax.ShapeDtypeStruct((B, D), jnp.float32),
              jax.ShapeDtypeStruct((B, D), jnp.float32)),
    mesh=_sc_mesh(),
    scratch_types=(
        pltpu.VMEM((BPW,), jnp.int32),
        pltpu.VMEM((BPW, D), jnp.float32),
        pltpu.VMEM((BPW, D), jnp.float32),
        pltpu.SemaphoreType.DMA,
    ),
    compiler_params=pltpu.CompilerParams(needs_layout_passes=False),
  )


def _select_call(*args):
    return _select_call_build()(*args)


def _dotf(a, b):
    return jnp.dot(a, b, preferred_element_type=jnp.float32)


def _elu(x):
    return jnp.where(x > 0.0, x, jnp.exp(jnp.minimum(x, 0.0)) - 1.0)


def _gat_prep_kernel(xs_ref, xt_ref, ws_ref, wt_ref, a_ref,
                     s1_ref, s2_ref, hj_ref):
    a1 = a_ref[0:D, :]
    a2 = a_ref[D:2 * D, :]
    w1 = _dotf(ws_ref[...], a1)
    hj = _dotf(xt_ref[...], wt_ref[...])
    hj_ref[...] = hj
    s2_ref[...] = _dotf(hj, a2)
    s1_ref[...] = _dotf(xs_ref[...], w1)


def _gat_prep(x_src, x_tgt, p):
    return pl.pallas_call(
        _gat_prep_kernel,
        grid=(NBLK,),
        in_specs=[
            pl.BlockSpec((RB, D), lambda i: (i, 0)),
            pl.BlockSpec((RB, D), lambda i: (i, 0)),
            pl.BlockSpec((D, D), lambda i: (0, 0)),
            pl.BlockSpec((D, D), lambda i: (0, 0)),
            pl.BlockSpec((2 * D, 1), lambda i: (0, 0)),
        ],
        out_specs=[
            pl.BlockSpec((RB, 1), lambda i: (i, 0)),
            pl.BlockSpec((RB, 1), lambda i: (i, 0)),
            pl.BlockSpec((RB, D), lambda i: (i, 0)),
        ],
        out_shape=[
            jax.ShapeDtypeStruct((NP, 1), jnp.float32),
            jax.ShapeDtypeStruct((NP, 1), jnp.float32),
            jax.ShapeDtypeStruct((NP, D), jnp.float32),
        ],
    )(x_src, x_tgt, p['W_src'], p['W_tgt'], p['a'])


def _gat_combine_kernel(accr_ref, accd_ref, out_ref):
    s = accr_ref[0] + accr_ref[1]
    d = accd_ref[0] + accd_ref[1]
    out_ref[...] = _elu(s / (d + 1e-08))


def _gat_combine(accr, accd):
    return pl.pallas_call(
        _gat_combine_kernel,
        grid=(NBLK,),
        in_specs=[
            pl.BlockSpec((NC, RB, D), lambda i: (0, i, 0)),
            pl.BlockSpec((NC, RB, 1), lambda i: (0, i, 0)),
        ],
        out_specs=pl.BlockSpec((RB, D), lambda i: (i, 0)),
        out_shape=jax.ShapeDtypeStruct((NP, D), jnp.float32),
    )(accr, accd.reshape(NC, NP, 1))


def _hg_a_kernel(x_ref, hyper_ref, w_ref, s_ref, t_ref):
    i = pl.program_id(0)
    x = x_ref[...]
    s = _dotf(x, hyper_ref[...])
    s_ref[...] = s
    y = _dotf(x, w_ref[...])
    tt = lax.dot_general(s, y, (((0,), (0,)), ((), ())),
                         preferred_element_type=jnp.float32)

    @pl.when(i == 0)
    def _():
        t_ref[...] = tt

    @pl.when(i > 0)
    def _():
        t_ref[...] += tt


def _hg_b_kernel(s_ref, t_ref, out_ref):
    out_ref[...] = _elu(_dotf(s_ref[...], t_ref[...]))


def _hg(x, hyper, w):
    s, t = pl.pallas_call(
        _hg_a_kernel,
        grid=(NBLK,),
        in_specs=[
            pl.BlockSpec((RB, D), lambda i: (i, 0)),
            pl.BlockSpec((D, D), lambda i: (0, 0)),
            pl.BlockSpec((D, D), lambda i: (0, 0)),
        ],
        out_specs=[
            pl.BlockSpec((RB, D), lambda i: (i, 0)),
            pl.BlockSpec((D, D), lambda i: (0, 0)),
        ],
        out_shape=[
            jax.ShapeDtypeStruct((NP, D), jnp.float32),
            jax.ShapeDtypeStruct((D, D), jnp.float32),
        ],
        compiler_params=pltpu.CompilerParams(
            dimension_semantics=("arbitrary",)),
    )(x, hyper, w)
    return pl.pallas_call(
        _hg_b_kernel,
        grid=(NBLK,),
        in_specs=[
            pl.BlockSpec((RB, D), lambda i: (i, 0)),
            pl.BlockSpec((D, D), lambda i: (0, 0)),
        ],
        out_specs=pl.BlockSpec((RB, D), lambda i: (i, 0)),
        out_shape=jax.ShapeDtypeStruct((NP, D), jnp.float32),
    )(s, t)


def _norm_kernel(z1_ref, z2_ref, z1n_ref, z2n_ref, pos_ref):
    z1 = z1_ref[...]
    z2 = z2_ref[...]
    n1 = jnp.sqrt(jnp.sum(z1 * z1, axis=1, keepdims=True))
    n2 = jnp.sqrt(jnp.sum(z2 * z2, axis=1, keepdims=True))
    z1n = z1 / (n1 + 1e-12)
    z2n = z2 / (n2 + 1e-12)
    z1n_ref[...] = z1n
    z2n_ref[...] = z2n
    pos_ref[...] = jnp.sum(z1n * z2n, axis=1, keepdims=True) / TEMP


def _nce_kernel(z1n_ref, z2n_ref, pos_ref, out_ref, acc_ref):
    r = pl.program_id(0)
    c = pl.program_id(1)
    sim = lax.dot_general(z1n_ref[...], z2n_ref[...],
                          (((1,), (1,)), ((), ())),
                          preferred_element_type=jnp.float32) / TEMP
    e = jnp.exp(sim)
    colid = c * RB + lax.broadcasted_iota(jnp.int32, (RB, RB), 1)
    e = jnp.where(colid < N, e, 0.0)
    rowsum = jnp.sum(e, axis=1, keepdims=True)

    @pl.when(c == 0)
    def _():
        acc_ref[...] = rowsum

    @pl.when(c > 0)
    def _():
        acc_ref[...] += rowsum

    @pl.when(c == NBLK - 1)
    def _():
        lse = jnp.log(acc_ref[...])
        contrib = lse - pos_ref[...]
        rowid = r * RB + lax.broadcasted_iota(jnp.int32, (RB, 1), 0)
        contrib = jnp.where(rowid < N, contrib, 0.0)
        out_ref[...] = jnp.sum(contrib).reshape(1, 1, 1)


def _info_nce(z1, z2):
    z1n, z2n, pos = pl.pallas_call(
        _norm_kernel,
        grid=(NBLK,),
        in_specs=[
            pl.BlockSpec((RB, D), lambda i: (i, 0)),
            pl.BlockSpec((RB, D), lambda i: (i, 0)),
        ],
        out_specs=[
            pl.BlockSpec((RB, D), lambda i: (i, 0)),
            pl.BlockSpec((RB, D), lambda i: (i, 0)),
            pl.BlockSpec((RB, 1), lambda i: (i, 0)),
        ],
        out_shape=[
            jax.ShapeDtypeStruct((NP, D), jnp.float32),
            jax.ShapeDtypeStruct((NP, D), jnp.float32),
            jax.ShapeDtypeStruct((NP, 1), jnp.float32),
        ],
    )(z1, z2)
    parts = pl.pallas_call(
        _nce_kernel,
        grid=(NBLK, NBLK),
        in_specs=[
            pl.BlockSpec((RB, D), lambda r, c: (r, 0)),
            pl.BlockSpec((RB, D), lambda r, c: (c, 0)),
            pl.BlockSpec((RB, 1), lambda r, c: (r, 0)),
        ],
        out_specs=pl.BlockSpec((1, 1, 1), lambda r, c: (r, 0, 0)),
        out_shape=jax.ShapeDtypeStruct((NBLK, 1, 1), jnp.float32),
        scratch_shapes=[pltpu.VMEM((RB, 1), jnp.float32)],
        compiler_params=pltpu.CompilerParams(
            dimension_semantics=("arbitrary", "arbitrary")),
    )(z1n, z2n, pos)
    return jnp.sum(parts) / N


def _head_kernel(st_ref, sd_ref, w1_ref, b1_ref, w2_ref, b2_ref,
                 gw_ref, gb_ref, fus_ref, gmf_ref, mlp_ref):
    st = st_ref[...]
    sd = sd_ref[...]
    gmf_vec = st * sd
    gmf = jnp.sum(gmf_vec, axis=1, keepdims=True)
    hid = _dotf(st, w1_ref[0:D, :]) + _dotf(sd, w1_ref[D:2 * D, :])
    hid = jnp.maximum(hid + b1_ref[...], 0.0)
    mlp = _dotf(hid, w2_ref[...]) + b2_ref[...]
    gate = 1.0 / (1.0 + jnp.exp(-(_dotf(gmf_vec, gw_ref[...]) + gb_ref[...])))
    fus_ref[...] = gate * gmf + (1.0 - gate) * mlp
    gmf_ref[...] = gmf
    mlp_ref[...] = mlp


def _head(sel_t, sel_d, p):
    hb = B // 4
    return pl.pallas_call(
        _head_kernel,
        grid=(4,),
        in_specs=[
            pl.BlockSpec((hb, D), lambda i: (i, 0)),
            pl.BlockSpec((hb, D), lambda i: (i, 0)),
            pl.BlockSpec((2 * D, D), lambda i: (0, 0)),
            pl.BlockSpec((1, D), lambda i: (0, 0)),
            pl.BlockSpec((D, 1), lambda i: (0, 0)),
            pl.BlockSpec((1, 1), lambda i: (0, 0)),
            pl.BlockSpec((D, 1), lambda i: (0, 0)),
            pl.BlockSpec((1, 1), lambda i: (0, 0)),
        ],
        out_specs=[
            pl.BlockSpec((hb, 1), lambda i: (i, 0)),
            pl.BlockSpec((hb, 1), lambda i: (i, 0)),
            pl.BlockSpec((hb, 1), lambda i: (i, 0)),
        ],
        out_shape=[
            jax.ShapeDtypeStruct((B, 1), jnp.float32),
            jax.ShapeDtypeStruct((B, 1), jnp.float32),
            jax.ShapeDtypeStruct((B, 1), jnp.float32),
        ],
    )(sel_t, sel_d, p['mlp_W1'], p['mlp_b1'].reshape(1, D),
      p['mlp_W2'], p['mlp_b2'].reshape(1, 1),
      p['gate_W'], p['gate_b'].reshape(1, 1))


def _pad_rows(x):
    return jnp.pad(x, ((0, NP - N), (0, 0)))


def _gat(x_src, x_tgt, pk, p, zr, zd):
    s1, s2, hj = _gat_prep(x_src, x_tgt, p)
    accr, accd = _edge_call(pk, s1.reshape(NP), s2.reshape(NP), hj, zr, zd)
    return _gat_combine(accr, accd)


def kernel(params, adj_src, adj_dst, trna_ids, disease_ids):
    p = params
    asrc = adj_src.astype(jnp.int32)
    adst = adj_dst.astype(jnp.int32)
    pk_t = ((asrc << 16) | adst).reshape(E // CHUNK, CHUNK)
    pk_d = ((adst << 16) | asrc).reshape(E // CHUNK, CHUNK)
    zr = jnp.zeros((SLAB, D), jnp.float32)
    zd = jnp.zeros((SLAB,), jnp.float32)

    sg_t = _pad_rows(p['trna_embed'])
    sg_d = _pad_rows(p['disease_embed'])

    sg_t = _gat(sg_t, sg_d, pk_t, p['gat_t'][0], zr, zd)
    sg_d = _gat(sg_d, sg_t, pk_d, p['gat_d'][0], zr, zd)

    hg_t = _hg(sg_t, p['thyper'], p['hg_t'][0])
    hg_d = _hg(sg_d, p['dhyper'], p['hg_d'][0])
    hg_t = _hg(hg_t, p['thyper'], p['hg_t'][1])
    hg_d = _hg(hg_d, p['dhyper'], p['hg_d'][1])

    sg_t = _gat(hg_t, hg_d, pk_t, p['gat_t'][1], zr, zd)
    sg_d = _gat(hg_d, hg_t, pk_d, p['gat_d'][1], zr, zd)
    sg_t = _gat(sg_t, sg_d, pk_t, p['gat_t'][2], zr, zd)
    sg_d = _gat(sg_d, sg_t, pk_d, p['gat_d'][2], zr, zd)

    hg_t2 = _hg(hg_t, p['thyper'], p['hg_t'][2])
    hg_d2 = _hg(hg_d, p['dhyper'], p['hg_d'][2])

    contrast = _info_nce(sg_t, hg_t2) + _info_nce(sg_d, hg_d2)

    sel_t, sel_d = _select_call(sg_t, hg_t2, sg_d, hg_d2,
                                trna_ids.astype(jnp.int32),
                                disease_ids.astype(jnp.int32))
    fus, gmf, mlp = _head(sel_t, sel_d, p)

    return (fus[:, 0], sg_t[:N], sg_d[:N], gmf[:, 0], mlp[:, 0], contrast)
